# baseline (speedup 1.0000x reference)
import functools

import jax
import jax.numpy as jnp
from jax import lax
from jax.experimental import pallas as pl
from jax.experimental.pallas import tpu as pltpu
from jax.experimental.pallas import tpu_sc as plsc

N = 10000
E = 320000
INDIM = 128
HID = 128
OUT = 40
OUTP = 128
EPS = 1e-5

NC = 2
NS = 16
CHUNK = 125
IGROUP = 8
PGROUP = 2
EROWS = E // CHUNK
RPW = EROWS // (NC * NS)
NIG = RPW // IGROUP
NROWS = 10240
RPS = NROWS // NS
ZROWS = 64


def _make_deg():
  mesh = plsc.VectorSubcoreMesh(
      core_axis_name="c", subcore_axis_name="s", num_cores=NC, num_subcores=NS)

  @functools.partial(
      pl.kernel,
      out_type=(jax.ShapeDtypeStruct((NROWS,), jnp.float32),
                jax.ShapeDtypeStruct((NROWS,), jnp.float32)),
      mesh=mesh,
      scratch_types=[
          pltpu.VMEM_SHARED((NROWS,), jnp.float32),
          pltpu.VMEM((RPS,), jnp.float32),
          pltpu.VMEM((CHUNK,), jnp.float32),
          pltpu.VMEM((IGROUP, CHUNK), jnp.int32),
      ],
  )
  def deg_kernel(dst_hbm, out0, out1, acc, zb, ones, didx):
    c = lax.axis_index("c")
    s = lax.axis_index("s")

    zeros16 = jnp.zeros((16,), jnp.float32)
    ones16 = jnp.ones((16,), jnp.float32)

    def fill(i, _):
      zb[pl.ds(i * 16, 16)] = zeros16
      return 0
    lax.fori_loop(0, RPS // 16, fill, 0)

    def fill1(i, _):
      ones[pl.ds(i * 16, 16)] = ones16
      return 0
    lax.fori_loop(0, CHUNK // 16, fill1, 0)
    ones[pl.ds(CHUNK - 16, 16)] = ones16

    pltpu.sync_copy(zb, acc.at[pl.ds(s * RPS, RPS)])
    plsc.subcore_barrier()

    row0 = (c * NS + s) * RPW

    def group_body(g, _):
      pltpu.sync_copy(dst_hbm.at[pl.ds(row0 + g * IGROUP, IGROUP), :], didx)
      for j in range(IGROUP):
        pltpu.sync_copy(ones, acc.at[didx.at[j]], add=True)
      return 0
    lax.fori_loop(0, NIG, group_body, 0)

    plsc.subcore_barrier()

    @pl.when(c == 0)
    def _():
      pltpu.sync_copy(acc.at[pl.ds(s * RPS, RPS)], out0.at[pl.ds(s * RPS, RPS)])

    @pl.when(c == 1)
    def _():
      pltpu.sync_copy(acc.at[pl.ds(s * RPS, RPS)], out1.at[pl.ds(s * RPS, RPS)])

  return deg_kernel


def _make_prop(fp):
  mesh = plsc.VectorSubcoreMesh(
      core_axis_name="c", subcore_axis_name="s", num_cores=NC, num_subcores=NS)

  @functools.partial(
      pl.kernel,
      out_type=(jax.ShapeDtypeStruct((NROWS, fp), jnp.float32),
                jax.ShapeDtypeStruct((NROWS, fp), jnp.float32)),
      mesh=mesh,
      scratch_types=[
          pltpu.VMEM_SHARED((NROWS, fp), jnp.float32),
          pltpu.VMEM((ZROWS, fp), jnp.float32),
          pltpu.VMEM((IGROUP, CHUNK), jnp.int32),
          pltpu.VMEM((IGROUP, CHUNK), jnp.int32),
          pltpu.VMEM((2, CHUNK, fp), jnp.float32),
          pltpu.SemaphoreType.DMA((2,)),
          pltpu.SemaphoreType.DMA((2,)),
      ],
  )
  def prop_kernel(hs_hbm, src_hbm, dst_hbm, out0, out1,
                  acc, zb, sidx, didx, rows, gsem, ssem):
    c = lax.axis_index("c")
    s = lax.axis_index("s")

    zeros16 = jnp.zeros((16,), jnp.float32)
    lanes = fp // 16

    def fill(i, _):
      zb[i // lanes, pl.ds((i % lanes) * 16, 16)] = zeros16
      return 0
    lax.fori_loop(0, ZROWS * lanes, fill, 0)

    for t in range(RPS // ZROWS):
      pltpu.sync_copy(zb, acc.at[pl.ds(s * RPS + t * ZROWS, ZROWS), :])
    plsc.subcore_barrier()

    row0 = (c * NS + s) * RPW

    def group_body(g, _):
      base = row0 + g * IGROUP
      pltpu.sync_copy(src_hbm.at[pl.ds(base, IGROUP), :], sidx)
      pltpu.sync_copy(dst_hbm.at[pl.ds(base, IGROUP), :], didx)
      gathers = [None] * IGROUP
      scatters = [None] * IGROUP
      gathers[0] = pltpu.async_copy(
          hs_hbm.at[sidx.at[0]], rows.at[0], gsem.at[0])
      for j in range(IGROUP):
        p = j % 2
        if j >= 1:
          scatters[j - 1].wait()
        if j + 1 < IGROUP:
          gathers[j + 1] = pltpu.async_copy(
              hs_hbm.at[sidx.at[j + 1]], rows.at[(j + 1) % 2],
              gsem.at[(j + 1) % 2])
        gathers[j].wait()
        scatters[j] = pltpu.async_copy(
            rows.at[p], acc.at[didx.at[j]], ssem.at[p], add=True)
      scatters[IGROUP - 1].wait()
      return 0
    lax.fori_loop(0, NIG, group_body, 0)

    plsc.subcore_barrier()

    @pl.when(c == 0)
    def _():
      pltpu.sync_copy(acc.at[pl.ds(s * RPS, RPS), :],
                      out0.at[pl.ds(s * RPS, RPS), :])

    @pl.when(c == 1)
    def _():
      pltpu.sync_copy(acc.at[pl.ds(s * RPS, RPS), :],
                      out1.at[pl.ds(s * RPS, RPS), :])

  return prop_kernel


_MM = dict(preferred_element_type=jnp.float32, precision=lax.Precision.HIGHEST)


def _tc_mm(x, w, mask, interpret=False):
  def body(x_ref, w_ref, m_ref, out_ref):
    out_ref[...] = jnp.dot(x_ref[...], w_ref[...] * m_ref[...], **_MM)

  return pl.pallas_call(
      body,
      out_shape=jax.ShapeDtypeStruct((N, HID), jnp.float32),
      interpret=interpret,
  )(x, w, mask)


def _tc_scale(h, d0, d1, interpret=False):
  def body(h_ref, d0_ref, d1_ref, out_ref):
    dis = lax.rsqrt(d0_ref[0:N] + d1_ref[0:N] + 1.0)
    out_ref[...] = h_ref[...] * dis

  return pl.pallas_call(
      body,
      out_shape=jax.ShapeDtypeStruct((N, HID), jnp.float32),
      interpret=interpret,
  )(h, d0, d1)


def _tc_mid(pa, pb, hsp, d0, d1, b, gamma, beta, w, mask, fout,
            interpret=False):
  def body(pa_ref, pb_ref, hs_ref, d0_ref, d1_ref, b_ref, g_ref, be_ref,
           w_ref, m_ref, out_ref):
    dis = lax.rsqrt(d0_ref[0:N] + d1_ref[0:N] + 1.0)
    t = (pa_ref[0:N] + pb_ref[0:N] + hs_ref[...]) * dis + b_ref[...]
    mean = jnp.mean(t, axis=0, keepdims=True)
    var = jnp.mean((t - mean) ** 2, axis=0, keepdims=True)
    y = (t - mean) * lax.rsqrt(var + EPS) * g_ref[...] + be_ref[...]
    y = jnp.maximum(y, 0.0)
    h = jnp.dot(y, w_ref[...] * m_ref[...], **_MM)
    out_ref[...] = h * dis

  return pl.pallas_call(
      body,
      out_shape=jax.ShapeDtypeStruct((N, fout), jnp.float32),
      interpret=interpret,
  )(pa, pb, hsp, d0, d1, b, gamma, beta, w, mask)


def _tc_out(pa, pb, hsp, d0, d1, b3, interpret=False):
  def body(pa_ref, pb_ref, hs_ref, d0_ref, d1_ref, b_ref, out_ref):
    dis = lax.rsqrt(d0_ref[0:N] + d1_ref[0:N] + 1.0)
    t = (pa_ref[0:N] + pb_ref[0:N] + hs_ref[...]) * dis
    logits = t[:, :OUT] + b_ref[...]
    m = jnp.max(logits, axis=1, keepdims=True)
    lse = jnp.log(jnp.sum(jnp.exp(logits - m), axis=1, keepdims=True)) + m
    out_ref[...] = logits - lse

  return pl.pallas_call(
      body,
      out_shape=jax.ShapeDtypeStruct((N, OUT), jnp.float32),
      interpret=interpret,
  )(pa, pb, hsp, d0, d1, b3)


def kernel(x, adj_t, W1, b1, gamma1, beta1, mask1, W2, b2, gamma2, beta2,
           mask2, W3, b3, mask3):
  src2d = adj_t[0].reshape(EROWS, CHUNK)
  dst2d = adj_t[1].reshape(EROWS, CHUNK)

  h1 = _tc_mm(x, W1, mask1)
  d0, d1 = _make_deg()(dst2d)
  d0r = d0.reshape(NROWS, 1)
  d1r = d1.reshape(NROWS, 1)

  b1r = b1.reshape(1, HID)
  g1r = gamma1.reshape(1, HID)
  be1r = beta1.reshape(1, HID)
  b2r = b2.reshape(1, HID)
  g2r = gamma2.reshape(1, HID)
  be2r = beta2.reshape(1, HID)
  b3r = b3.reshape(1, OUT)
  W3p = jnp.pad(W3, ((0, 0), (0, OUTP - OUT)))
  mask3p = jnp.pad(mask3, ((0, 0), (0, OUTP - OUT)))

  prop128 = _make_prop(HID)

  hs1 = _tc_scale(h1, d0r, d1r)
  p1a, p1b = prop128(hs1, src2d, dst2d)
  hs2 = _tc_mid(p1a, p1b, hs1, d0r, d1r, b1r, g1r, be1r, W2, mask2, HID)
  p2a, p2b = prop128(hs2, src2d, dst2d)
  hs3 = _tc_mid(p2a, p2b, hs2, d0r, d1r, b2r, g2r, be2r, W3p, mask3p, OUTP)
  p3a, p3b = prop128(hs3, src2d, dst2d)
  return _tc_out(p3a, p3b, hs3, d0r, d1r, b3r)


# speedup vs baseline: 1.1588x; 1.1588x over previous
import functools

import jax
import jax.numpy as jnp
from jax import lax
from jax.experimental import pallas as pl
from jax.experimental.pallas import tpu as pltpu
from jax.experimental.pallas import tpu_sc as plsc

N = 10000
E = 320000
INDIM = 128
HID = 128
OUT = 40
OUTP = 128
EPS = 1e-5

NC = 2
NS = 16
CHUNK = 125
IGROUP = 8
PGROUP = 2
EROWS = E // CHUNK
RPW = EROWS // (NC * NS)
NIG = RPW // IGROUP
NROWS = 10240
RPS = NROWS // NS
ZROWS = 64


def _make_deg():
  mesh = plsc.VectorSubcoreMesh(
      core_axis_name="c", subcore_axis_name="s", num_cores=NC, num_subcores=NS)

  @functools.partial(
      pl.kernel,
      out_type=(jax.ShapeDtypeStruct((NROWS,), jnp.float32),
                jax.ShapeDtypeStruct((NROWS,), jnp.float32)),
      mesh=mesh,
      scratch_types=[
          pltpu.VMEM_SHARED((NROWS,), jnp.float32),
          pltpu.VMEM((RPS,), jnp.float32),
          pltpu.VMEM((CHUNK,), jnp.float32),
          pltpu.VMEM((IGROUP, CHUNK), jnp.int32),
      ],
  )
  def deg_kernel(dst_hbm, out0, out1, acc, zb, ones, didx):
    c = lax.axis_index("c")
    s = lax.axis_index("s")

    zeros16 = jnp.zeros((16,), jnp.float32)
    ones16 = jnp.ones((16,), jnp.float32)

    def fill(i, _):
      zb[pl.ds(i * 16, 16)] = zeros16
      return 0
    lax.fori_loop(0, RPS // 16, fill, 0)

    def fill1(i, _):
      ones[pl.ds(i * 16, 16)] = ones16
      return 0
    lax.fori_loop(0, CHUNK // 16, fill1, 0)
    ones[pl.ds(CHUNK - 16, 16)] = ones16

    pltpu.sync_copy(zb, acc.at[pl.ds(s * RPS, RPS)])
    plsc.subcore_barrier()

    row0 = (c * NS + s) * RPW

    def group_body(g, _):
      pltpu.sync_copy(dst_hbm.at[pl.ds(row0 + g * IGROUP, IGROUP), :], didx)
      for j in range(IGROUP):
        pltpu.sync_copy(ones, acc.at[didx.at[j]], add=True)
      return 0
    lax.fori_loop(0, NIG, group_body, 0)

    plsc.subcore_barrier()

    @pl.when(c == 0)
    def _():
      pltpu.sync_copy(acc.at[pl.ds(s * RPS, RPS)], out0.at[pl.ds(s * RPS, RPS)])

    @pl.when(c == 1)
    def _():
      pltpu.sync_copy(acc.at[pl.ds(s * RPS, RPS)], out1.at[pl.ds(s * RPS, RPS)])

  return deg_kernel


def _make_prop(fp):
  mesh = plsc.VectorSubcoreMesh(
      core_axis_name="c", subcore_axis_name="s", num_cores=NC, num_subcores=NS)

  @functools.partial(
      pl.kernel,
      out_type=(jax.ShapeDtypeStruct((NROWS, fp), jnp.float32),
                jax.ShapeDtypeStruct((NROWS, fp), jnp.float32)),
      mesh=mesh,
      scratch_types=[
          pltpu.VMEM_SHARED((NROWS, fp), jnp.float32),
          pltpu.VMEM((ZROWS, fp), jnp.float32),
          pltpu.VMEM((2, IGROUP, CHUNK), jnp.int32),
          pltpu.VMEM((2, IGROUP, CHUNK), jnp.int32),
          pltpu.VMEM((2, CHUNK, fp), jnp.float32),
          pltpu.SemaphoreType.DMA((2,)),
          pltpu.SemaphoreType.DMA((2,)),
          pltpu.SemaphoreType.DMA((2,)),
      ],
  )
  def prop_kernel(hs_hbm, src_hbm, dst_hbm, out0, out1,
                  acc, zb, sidx, didx, rows, gsem, ssem, isem):
    c = lax.axis_index("c")
    s = lax.axis_index("s")
    row0 = (c * NS + s) * RPW

    def gather_start(ib, j, p):
      pltpu.async_copy(hs_hbm.at[sidx.at[ib, j]], rows.at[p], gsem.at[p])

    def gather_wait(p):
      pltpu.make_async_copy(
          hs_hbm.at[sidx.at[0, 0]], rows.at[p], gsem.at[p]).wait()

    def scatter_start(ib, j, p):
      pltpu.async_copy(rows.at[p], acc.at[didx.at[ib, j]], ssem.at[p],
                       add=True)

    def scatter_wait(p):
      pltpu.make_async_copy(
          rows.at[p], acc.at[didx.at[0, 0]], ssem.at[p]).wait()

    pltpu.sync_copy(src_hbm.at[pl.ds(row0, IGROUP), :], sidx.at[0])
    pltpu.sync_copy(dst_hbm.at[pl.ds(row0, IGROUP), :], didx.at[0])
    gather_start(0, 0, 0)

    zeros16 = jnp.zeros((16,), jnp.float32)
    lanes = fp // 16

    def fill(i, _):
      zb[i // lanes, pl.ds((i % lanes) * 16, 16)] = zeros16
      return 0
    lax.fori_loop(0, ZROWS * lanes, fill, 0)

    for t in range(RPS // ZROWS):
      pltpu.sync_copy(zb, acc.at[pl.ds(s * RPS + t * ZROWS, ZROWS), :])
    plsc.subcore_barrier()

    pltpu.async_copy(rows.at[1], acc.at[pl.ds(NROWS - 128, CHUNK), :],
                     ssem.at[1])

    def group_body(g, _):
      ib = g % 2
      inext = (g + 1) % 2

      @pl.when(g < NIG - 1)
      def _():
        nbase = row0 + (g + 1) * IGROUP
        pltpu.async_copy(src_hbm.at[pl.ds(nbase, IGROUP), :],
                         sidx.at[inext], isem.at[inext])
        pltpu.async_copy(dst_hbm.at[pl.ds(nbase, IGROUP), :],
                         didx.at[inext], isem.at[inext])

      for j in range(IGROUP):
        p = j % 2
        scatter_wait((j + 1) % 2)
        if j < IGROUP - 1:
          gather_start(ib, j + 1, (j + 1) % 2)
        else:
          @pl.when(g < NIG - 1)
          def _():
            pltpu.make_async_copy(src_hbm.at[pl.ds(0, IGROUP), :],
                                  sidx.at[inext], isem.at[inext]).wait()
            pltpu.make_async_copy(dst_hbm.at[pl.ds(0, IGROUP), :],
                                  didx.at[inext], isem.at[inext]).wait()
            gather_start(inext, 0, 0)
        gather_wait(p)
        scatter_start(ib, j, p)
      return 0
    lax.fori_loop(0, NIG, group_body, 0)

    scatter_wait(1)
    plsc.subcore_barrier()

    @pl.when(c == 0)
    def _():
      pltpu.sync_copy(acc.at[pl.ds(s * RPS, RPS), :],
                      out0.at[pl.ds(s * RPS, RPS), :])

    @pl.when(c == 1)
    def _():
      pltpu.sync_copy(acc.at[pl.ds(s * RPS, RPS), :],
                      out1.at[pl.ds(s * RPS, RPS), :])

  return prop_kernel


_MM = dict(preferred_element_type=jnp.float32, precision=lax.Precision.HIGHEST)


def _tc_mm(x, w, mask, interpret=False):
  def body(x_ref, w_ref, m_ref, out_ref):
    out_ref[...] = jnp.dot(x_ref[...], w_ref[...] * m_ref[...], **_MM)

  return pl.pallas_call(
      body,
      out_shape=jax.ShapeDtypeStruct((N, HID), jnp.float32),
      interpret=interpret,
  )(x, w, mask)


def _tc_scale(h, d0, d1, interpret=False):
  def body(h_ref, d0_ref, d1_ref, out_ref):
    dis = lax.rsqrt(d0_ref[0:N] + d1_ref[0:N] + 1.0)
    out_ref[...] = h_ref[...] * dis

  return pl.pallas_call(
      body,
      out_shape=jax.ShapeDtypeStruct((N, HID), jnp.float32),
      interpret=interpret,
  )(h, d0, d1)


def _tc_mid(pa, pb, hsp, d0, d1, b, gamma, beta, w, mask, fout,
            interpret=False):
  def body(pa_ref, pb_ref, hs_ref, d0_ref, d1_ref, b_ref, g_ref, be_ref,
           w_ref, m_ref, out_ref):
    dis = lax.rsqrt(d0_ref[0:N] + d1_ref[0:N] + 1.0)
    t = (pa_ref[0:N] + pb_ref[0:N] + hs_ref[...]) * dis + b_ref[...]
    mean = jnp.mean(t, axis=0, keepdims=True)
    var = jnp.mean((t - mean) ** 2, axis=0, keepdims=True)
    y = (t - mean) * lax.rsqrt(var + EPS) * g_ref[...] + be_ref[...]
    y = jnp.maximum(y, 0.0)
    h = jnp.dot(y, w_ref[...] * m_ref[...], **_MM)
    out_ref[...] = h * dis

  return pl.pallas_call(
      body,
      out_shape=jax.ShapeDtypeStruct((N, fout), jnp.float32),
      interpret=interpret,
  )(pa, pb, hsp, d0, d1, b, gamma, beta, w, mask)


def _tc_out(pa, pb, hsp, d0, d1, b3, interpret=False):
  def body(pa_ref, pb_ref, hs_ref, d0_ref, d1_ref, b_ref, out_ref):
    dis = lax.rsqrt(d0_ref[0:N] + d1_ref[0:N] + 1.0)
    t = (pa_ref[0:N] + pb_ref[0:N] + hs_ref[...]) * dis
    logits = t[:, :OUT] + b_ref[...]
    m = jnp.max(logits, axis=1, keepdims=True)
    lse = jnp.log(jnp.sum(jnp.exp(logits - m), axis=1, keepdims=True)) + m
    out_ref[...] = logits - lse

  return pl.pallas_call(
      body,
      out_shape=jax.ShapeDtypeStruct((N, OUT), jnp.float32),
      interpret=interpret,
  )(pa, pb, hsp, d0, d1, b3)


def kernel(x, adj_t, W1, b1, gamma1, beta1, mask1, W2, b2, gamma2, beta2,
           mask2, W3, b3, mask3):
  src2d = adj_t[0].reshape(EROWS, CHUNK)
  dst2d = adj_t[1].reshape(EROWS, CHUNK)

  h1 = _tc_mm(x, W1, mask1)
  d0, d1 = _make_deg()(dst2d)
  d0r = d0.reshape(NROWS, 1)
  d1r = d1.reshape(NROWS, 1)

  b1r = b1.reshape(1, HID)
  g1r = gamma1.reshape(1, HID)
  be1r = beta1.reshape(1, HID)
  b2r = b2.reshape(1, HID)
  g2r = gamma2.reshape(1, HID)
  be2r = beta2.reshape(1, HID)
  b3r = b3.reshape(1, OUT)
  W3p = jnp.pad(W3, ((0, 0), (0, OUTP - OUT)))
  mask3p = jnp.pad(mask3, ((0, 0), (0, OUTP - OUT)))

  prop128 = _make_prop(HID)

  hs1 = _tc_scale(h1, d0r, d1r)
  p1a, p1b = prop128(hs1, src2d, dst2d)
  hs2 = _tc_mid(p1a, p1b, hs1, d0r, d1r, b1r, g1r, be1r, W2, mask2, HID)
  p2a, p2b = prop128(hs2, src2d, dst2d)
  hs3 = _tc_mid(p2a, p2b, hs2, d0r, d1r, b2r, g2r, be2r, W3p, mask3p, OUTP)
  p3a, p3b = prop128(hs3, src2d, dst2d)
  return _tc_out(p3a, p3b, hs3, d0r, d1r, b3r)


# speedup vs baseline: 1.1652x; 1.0055x over previous
import functools

import jax
import jax.numpy as jnp
from jax import lax
from jax.experimental import pallas as pl
from jax.experimental.pallas import tpu as pltpu
from jax.experimental.pallas import tpu_sc as plsc

N = 10000
E = 320000
INDIM = 128
HID = 128
OUT = 40
OUTP = 128
EPS = 1e-5

NC = 2
NS = 16
CHUNK = 125
IGROUP = 8
PGROUP = 2
EROWS = E // CHUNK
RPW = EROWS // (NC * NS)
NIG = RPW // IGROUP
NROWS = 10240
RPS = NROWS // NS
ZROWS = 64


def _make_deg():
  mesh = plsc.VectorSubcoreMesh(
      core_axis_name="c", subcore_axis_name="s", num_cores=NC, num_subcores=NS)

  @functools.partial(
      pl.kernel,
      out_type=(jax.ShapeDtypeStruct((NROWS,), jnp.float32),
                jax.ShapeDtypeStruct((NROWS,), jnp.float32)),
      mesh=mesh,
      scratch_types=[
          pltpu.VMEM_SHARED((NROWS,), jnp.float32),
          pltpu.VMEM((RPS,), jnp.float32),
          pltpu.VMEM((CHUNK,), jnp.float32),
          pltpu.VMEM((2, IGROUP, CHUNK), jnp.int32),
          pltpu.SemaphoreType.DMA((2,)),
      ],
  )
  def deg_kernel(dst_hbm, out0, out1, acc, zb, ones, didx, ssem):
    c = lax.axis_index("c")
    s = lax.axis_index("s")

    zeros16 = jnp.zeros((16,), jnp.float32)
    ones16 = jnp.ones((16,), jnp.float32)

    def fill(i, _):
      zb[pl.ds(i * 16, 16)] = zeros16
      return 0
    lax.fori_loop(0, RPS // 16, fill, 0)

    def fill1(i, _):
      ones[pl.ds(i * 16, 16)] = ones16
      return 0
    lax.fori_loop(0, CHUNK // 16, fill1, 0)
    ones[pl.ds(CHUNK - 16, 16)] = ones16

    pltpu.sync_copy(zb, acc.at[pl.ds(s * RPS, RPS)])
    plsc.subcore_barrier()

    row0 = (c * NS + s) * RPW

    def fire(ib):
      for j in range(IGROUP):
        pltpu.async_copy(ones, acc.at[didx.at[ib, j]], ssem.at[ib], add=True)

    def drain(ib):
      for j in range(IGROUP):
        pltpu.make_async_copy(ones, acc.at[didx.at[0, 0]], ssem.at[ib]).wait()

    pltpu.sync_copy(dst_hbm.at[pl.ds(row0, IGROUP), :], didx.at[0])
    fire(0)

    def group_body(g, _):
      ib = g % 2
      pltpu.sync_copy(dst_hbm.at[pl.ds(row0 + g * IGROUP, IGROUP), :],
                      didx.at[ib])
      fire(ib)
      drain((g - 1) % 2)
      return 0
    lax.fori_loop(1, NIG, group_body, 0)

    drain((NIG - 1) % 2)
    plsc.subcore_barrier()

    @pl.when(c == 0)
    def _():
      pltpu.sync_copy(acc.at[pl.ds(s * RPS, RPS)], out0.at[pl.ds(s * RPS, RPS)])

    @pl.when(c == 1)
    def _():
      pltpu.sync_copy(acc.at[pl.ds(s * RPS, RPS)], out1.at[pl.ds(s * RPS, RPS)])

  return deg_kernel


def _make_prop(fp):
  mesh = plsc.VectorSubcoreMesh(
      core_axis_name="c", subcore_axis_name="s", num_cores=NC, num_subcores=NS)

  @functools.partial(
      pl.kernel,
      out_type=(jax.ShapeDtypeStruct((NROWS, fp), jnp.float32),
                jax.ShapeDtypeStruct((NROWS, fp), jnp.float32)),
      mesh=mesh,
      scratch_types=[
          pltpu.VMEM_SHARED((NROWS, fp), jnp.float32),
          pltpu.VMEM((ZROWS, fp), jnp.float32),
          pltpu.VMEM((2, IGROUP, CHUNK), jnp.int32),
          pltpu.VMEM((2, IGROUP, CHUNK), jnp.int32),
          pltpu.VMEM((2, CHUNK, fp), jnp.float32),
          pltpu.SemaphoreType.DMA((2,)),
          pltpu.SemaphoreType.DMA((2,)),
          pltpu.SemaphoreType.DMA((2,)),
      ],
  )
  def prop_kernel(hs_hbm, src_hbm, dst_hbm, out0, out1,
                  acc, zb, sidx, didx, rows, gsem, ssem, isem):
    c = lax.axis_index("c")
    s = lax.axis_index("s")
    row0 = (c * NS + s) * RPW

    def gather_start(ib, j, p):
      pltpu.async_copy(hs_hbm.at[sidx.at[ib, j]], rows.at[p], gsem.at[p])

    def gather_wait(p):
      pltpu.make_async_copy(
          hs_hbm.at[sidx.at[0, 0]], rows.at[p], gsem.at[p]).wait()

    def scatter_start(ib, j, p):
      pltpu.async_copy(rows.at[p], acc.at[didx.at[ib, j]], ssem.at[p],
                       add=True)

    def scatter_wait(p):
      pltpu.make_async_copy(
          rows.at[p], acc.at[didx.at[0, 0]], ssem.at[p]).wait()

    pltpu.sync_copy(src_hbm.at[pl.ds(row0, IGROUP), :], sidx.at[0])
    pltpu.sync_copy(dst_hbm.at[pl.ds(row0, IGROUP), :], didx.at[0])
    gather_start(0, 0, 0)

    zeros16 = jnp.zeros((16,), jnp.float32)
    lanes = fp // 16

    def fill(i, _):
      zb[i // lanes, pl.ds((i % lanes) * 16, 16)] = zeros16
      return 0
    lax.fori_loop(0, ZROWS * lanes, fill, 0)

    @pl.when(c == 0)
    def _():
      @pl.when(s < NS - 1)
      def _():
        pltpu.sync_copy(hs_hbm.at[pl.ds(s * RPS, RPS), :],
                        acc.at[pl.ds(s * RPS, RPS), :])

      @pl.when(s == NS - 1)
      def _():
        pltpu.sync_copy(hs_hbm.at[pl.ds(N - 400, 400), :],
                        acc.at[pl.ds(NROWS - RPS, 400), :])
        for t in range(3):
          pltpu.sync_copy(zb, acc.at[pl.ds(N + t * ZROWS, ZROWS), :])
        pltpu.sync_copy(zb.at[pl.ds(0, 48), :],
                        acc.at[pl.ds(N + 3 * ZROWS, 48), :])

    @pl.when(c == 1)
    def _():
      for t in range(RPS // ZROWS):
        pltpu.sync_copy(zb, acc.at[pl.ds(s * RPS + t * ZROWS, ZROWS), :])

    plsc.subcore_barrier()

    pltpu.async_copy(rows.at[1], acc.at[pl.ds(NROWS - 128, CHUNK), :],
                     ssem.at[1])

    def group_body(g, _):
      ib = g % 2
      inext = (g + 1) % 2

      @pl.when(g < NIG - 1)
      def _():
        nbase = row0 + (g + 1) * IGROUP
        pltpu.async_copy(src_hbm.at[pl.ds(nbase, IGROUP), :],
                         sidx.at[inext], isem.at[inext])
        pltpu.async_copy(dst_hbm.at[pl.ds(nbase, IGROUP), :],
                         didx.at[inext], isem.at[inext])

      for j in range(IGROUP):
        p = j % 2
        scatter_wait((j + 1) % 2)
        if j < IGROUP - 1:
          gather_start(ib, j + 1, (j + 1) % 2)
        else:
          @pl.when(g < NIG - 1)
          def _():
            pltpu.make_async_copy(src_hbm.at[pl.ds(0, IGROUP), :],
                                  sidx.at[inext], isem.at[inext]).wait()
            pltpu.make_async_copy(dst_hbm.at[pl.ds(0, IGROUP), :],
                                  didx.at[inext], isem.at[inext]).wait()
            gather_start(inext, 0, 0)
        gather_wait(p)
        scatter_start(ib, j, p)
      return 0
    lax.fori_loop(0, NIG, group_body, 0)

    scatter_wait(1)
    plsc.subcore_barrier()

    @pl.when(c == 0)
    def _():
      pltpu.sync_copy(acc.at[pl.ds(s * RPS, RPS), :],
                      out0.at[pl.ds(s * RPS, RPS), :])

    @pl.when(c == 1)
    def _():
      pltpu.sync_copy(acc.at[pl.ds(s * RPS, RPS), :],
                      out1.at[pl.ds(s * RPS, RPS), :])

  return prop_kernel


_MM = dict(preferred_element_type=jnp.float32, precision=lax.Precision.HIGHEST)


def _tc_mm(x, w, mask, interpret=False):
  def body(x_ref, w_ref, m_ref, out_ref):
    out_ref[...] = jnp.dot(x_ref[...], w_ref[...] * m_ref[...], **_MM)

  return pl.pallas_call(
      body,
      out_shape=jax.ShapeDtypeStruct((N, HID), jnp.float32),
      interpret=interpret,
  )(x, w, mask)


def _tc_scale(h, d0, d1, interpret=False):
  def body(h_ref, d0_ref, d1_ref, out_ref):
    dis = lax.rsqrt(d0_ref[0:N] + d1_ref[0:N] + 1.0)
    out_ref[...] = h_ref[...] * dis

  return pl.pallas_call(
      body,
      out_shape=jax.ShapeDtypeStruct((N, HID), jnp.float32),
      interpret=interpret,
  )(h, d0, d1)


def _tc_mid(pa, pb, d0, d1, b, gamma, beta, w, mask, fout, interpret=False):
  def body(pa_ref, pb_ref, d0_ref, d1_ref, b_ref, g_ref, be_ref,
           w_ref, m_ref, out_ref):
    dis = lax.rsqrt(d0_ref[0:N] + d1_ref[0:N] + 1.0)
    t = (pa_ref[0:N] + pb_ref[0:N]) * dis + b_ref[...]
    mean = jnp.mean(t, axis=0, keepdims=True)
    var = jnp.mean((t - mean) ** 2, axis=0, keepdims=True)
    y = (t - mean) * lax.rsqrt(var + EPS) * g_ref[...] + be_ref[...]
    y = jnp.maximum(y, 0.0)
    h = jnp.dot(y, w_ref[...] * m_ref[...], **_MM)
    out_ref[...] = h * dis

  return pl.pallas_call(
      body,
      out_shape=jax.ShapeDtypeStruct((N, fout), jnp.float32),
      interpret=interpret,
  )(pa, pb, d0, d1, b, gamma, beta, w, mask)


def _tc_out(pa, pb, d0, d1, b3, interpret=False):
  def body(pa_ref, pb_ref, d0_ref, d1_ref, b_ref, out_ref):
    dis = lax.rsqrt(d0_ref[0:N] + d1_ref[0:N] + 1.0)
    t = (pa_ref[0:N] + pb_ref[0:N]) * dis
    logits = t[:, :OUT] + b_ref[...]
    m = jnp.max(logits, axis=1, keepdims=True)
    lse = jnp.log(jnp.sum(jnp.exp(logits - m), axis=1, keepdims=True)) + m
    out_ref[...] = logits - lse

  return pl.pallas_call(
      body,
      out_shape=jax.ShapeDtypeStruct((N, OUT), jnp.float32),
      interpret=interpret,
  )(pa, pb, d0, d1, b3)


def kernel(x, adj_t, W1, b1, gamma1, beta1, mask1, W2, b2, gamma2, beta2,
           mask2, W3, b3, mask3):
  src2d = adj_t[0].reshape(EROWS, CHUNK)
  dst2d = adj_t[1].reshape(EROWS, CHUNK)

  h1 = _tc_mm(x, W1, mask1)
  d0, d1 = _make_deg()(dst2d)
  d0r = d0.reshape(NROWS, 1)
  d1r = d1.reshape(NROWS, 1)

  b1r = b1.reshape(1, HID)
  g1r = gamma1.reshape(1, HID)
  be1r = beta1.reshape(1, HID)
  b2r = b2.reshape(1, HID)
  g2r = gamma2.reshape(1, HID)
  be2r = beta2.reshape(1, HID)
  b3r = b3.reshape(1, OUT)
  W3p = jnp.pad(W3, ((0, 0), (0, OUTP - OUT)))
  mask3p = jnp.pad(mask3, ((0, 0), (0, OUTP - OUT)))

  prop128 = _make_prop(HID)

  hs1 = _tc_scale(h1, d0r, d1r)
  p1a, p1b = prop128(hs1, src2d, dst2d)
  hs2 = _tc_mid(p1a, p1b, d0r, d1r, b1r, g1r, be1r, W2, mask2, HID)
  p2a, p2b = prop128(hs2, src2d, dst2d)
  hs3 = _tc_mid(p2a, p2b, d0r, d1r, b2r, g2r, be2r, W3p, mask3p, OUTP)
  p3a, p3b = prop128(hs3, src2d, dst2d)
  return _tc_out(p3a, p3b, d0r, d1r, b3r)
